# Initial kernel scaffold; baseline (speedup 1.0000x reference)
import jax
import jax.numpy as jnp
from jax.experimental import pallas as pl


def kernel(question_embs, edge_index, edge_values):
    raise NotImplementedError("write your pallas kernel here")


# speedup vs baseline: 4.6302x; 4.6302x over previous
import jax
import jax.numpy as jnp
from jax import lax
from jax.experimental import pallas as pl
from jax.experimental.pallas import tpu as pltpu
from jax.experimental.pallas import tpu_sc as plsc

N = 10000
E = 320000
D = 128
NUM_LAYERS = 3

NC = 2
NS = 16
DH = D // NC

EPT = E // NS
CHUNK = 80
BLK = 10
NCH = EPT // CHUNK
NBLK = NCH // BLK

RPT = N // NS
RC = 125
NRC = RPT // RC


def _zeros16():
    return jnp.zeros((16,), dtype=jnp.float32)


def _body(x_hbm, dst_hbm, src_hbm, w_hbm, out_hbm,
          A, B, dstb, srcb, wb, msg, bufT, bufB, bufX, sem):
    core = lax.axis_index("c")
    sub = lax.axis_index("s")

    def zb(r, _):
        for cc in range(DH // 16):
            bufB[r, pl.ds(cc * 16, 16)] = _zeros16()
        return _
    lax.fori_loop(0, RC, zb, None)

    for k in range(NRC):
        r0 = sub * RPT + k * RC
        pltpu.sync_copy(x_hbm.at[core, pl.ds(r0, RC)], bufT)
        pltpu.sync_copy(bufT, A.at[pl.ds(r0, RC)])
        pltpu.sync_copy(bufB, B.at[pl.ds(r0, RC)])
    plsc.subcore_barrier()

    for layer in range(NUM_LAYERS):
        def edge_block(b, _):
            blk = pl.ds(b * BLK, BLK)
            pltpu.sync_copy(dst_hbm.at[sub, blk], dstb)
            pltpu.sync_copy(src_hbm.at[sub, blk], srcb)
            pltpu.sync_copy(w_hbm.at[sub, blk], wb)

            def edge_chunk(j, _):
                pltpu.async_copy(A.at[srcb.at[j]], msg, sem).wait()
                row_j = jnp.full((16,), j, dtype=jnp.int32)
                for e in range(CHUNK):
                    col_e = jnp.full((16,), e, dtype=jnp.int32)
                    wbc = plsc.load_gather(wb, [row_j, col_e])
                    for cc in range(DH // 16):
                        sl = (e, pl.ds(cc * 16, 16))
                        msg[sl] = msg[sl] * wbc
                pltpu.sync_copy(msg, B.at[dstb.at[j]], add=True)
                return _
            lax.fori_loop(0, BLK, edge_chunk, None)
            return _
        lax.fori_loop(0, NBLK, edge_block, None)
        plsc.subcore_barrier()

        last = layer == NUM_LAYERS - 1
        for k in range(NRC):
            r0 = sub * RPT + k * RC
            rows = pl.ds(r0, RC)
            pltpu.sync_copy(x_hbm.at[core, rows], bufX)
            pltpu.sync_copy(A.at[rows], bufT)
            pltpu.sync_copy(B.at[rows], bufB)

            if not last:
                def dense_row(r, _):
                    for cc in range(DH // 16):
                        sl = (r, pl.ds(cc * 16, 16))
                        bufT[sl] = bufX[sl] + bufT[sl] + bufB[sl]
                        bufB[sl] = _zeros16()
                    return _
                lax.fori_loop(0, RC, dense_row, None)
                pltpu.sync_copy(bufT, A.at[rows])
                pltpu.sync_copy(bufB, B.at[rows])
            else:
                def final_row(r, _):
                    for cc in range(DH // 16):
                        sl = (r, pl.ds(cc * 16, 16))
                        bufT[sl] = (bufX[sl] + bufT[sl] + bufB[sl]) * 0.25
                    return _
                lax.fori_loop(0, RC, final_row, None)
                pltpu.sync_copy(bufT, out_hbm.at[core, rows])
        if not last:
            plsc.subcore_barrier()


@jax.jit
def kernel(question_embs, edge_index, edge_values):
    xr = question_embs.reshape(N, NC, DH).transpose(1, 0, 2)
    dst_r = edge_index[0].reshape(NS, NCH, CHUNK)
    src_r = edge_index[1].reshape(NS, NCH, CHUNK)
    w_r = edge_values.reshape(NS, NCH, CHUNK)

    mesh = plsc.VectorSubcoreMesh(core_axis_name="c", subcore_axis_name="s")
    f = pl.kernel(
        _body,
        out_type=jax.ShapeDtypeStruct((NC, N, DH), jnp.float32),
        mesh=mesh,
        compiler_params=pltpu.CompilerParams(
            use_tc_tiling_on_sc=False, needs_layout_passes=False),
        scratch_types=[
            pltpu.VMEM_SHARED((N, DH), jnp.float32),
            pltpu.VMEM_SHARED((N, DH), jnp.float32),
            pltpu.VMEM((BLK, CHUNK), jnp.int32),
            pltpu.VMEM((BLK, CHUNK), jnp.int32),
            pltpu.VMEM((BLK, CHUNK), jnp.float32),
            pltpu.VMEM((CHUNK, DH), jnp.float32),
            pltpu.VMEM((RC, DH), jnp.float32),
            pltpu.VMEM((RC, DH), jnp.float32),
            pltpu.VMEM((RC, DH), jnp.float32),
            pltpu.SemaphoreType.DMA,
        ],
    )
    out_r = f(xr, dst_r, src_r, w_r)
    return out_r.transpose(1, 0, 2).reshape(N, D)
